# Initial kernel scaffold; baseline (speedup 1.0000x reference)
import jax
import jax.numpy as jnp
from jax.experimental import pallas as pl


def kernel(x, pos_embed_weight):
    raise NotImplementedError("write your pallas kernel here")


# speedup vs baseline: 5.5905x; 5.5905x over previous
import jax
import jax.numpy as jnp
from jax.experimental import pallas as pl

BLOCK_ROWS = 512


def _body(w_ref, o_ref):
    o_ref[...] = jnp.broadcast_to(w_ref[...][None], o_ref.shape)


def kernel(x, pos_embed_weight):
    bsz, seq_len = x.shape
    max_len, d_model = pos_embed_weight.shape
    n_blocks = seq_len // BLOCK_ROWS
    out = pl.pallas_call(
        _body,
        grid=(n_blocks,),
        in_specs=[pl.BlockSpec((BLOCK_ROWS, d_model), lambda i: (i, 0))],
        out_specs=pl.BlockSpec((bsz, BLOCK_ROWS, d_model), lambda i: (0, i, 0)),
        out_shape=jax.ShapeDtypeStruct((bsz, seq_len, d_model), jnp.float32),
    )(pos_embed_weight)
    return out
